# baseline (speedup 1.0000x reference)
import functools

import jax
import jax.numpy as jnp
from jax import lax
from jax.experimental import pallas as pl
from jax.experimental.pallas import tpu as pltpu
from jax.experimental.pallas import tpu_sc as plsc

_info = plsc.get_sparse_core_info()
_NC, _NS, _L = _info.num_cores, _info.num_subcores, _info.num_lanes
_NW = _NC * _NS

_N = 4 * 4096 * 2048
_PER_W = _N // _NW
_CHUNK = 16384
_CHUNKS = _PER_W // _CHUNK


def _sc_relu(x_hbm, o_hbm, buf):
    wid = lax.axis_index("s") * _NC + lax.axis_index("c")
    base = wid * _PER_W

    def chunk_body(ci, carry):
        off = base + ci * _CHUNK
        pltpu.sync_copy(x_hbm.at[pl.ds(off, _CHUNK)], buf)

        def vec_body(vi, c):
            s = pl.ds(vi * _L, _L)
            buf[s] = jnp.maximum(buf[s], 0.0)
            return c

        lax.fori_loop(0, _CHUNK // _L, vec_body, 0)
        pltpu.sync_copy(buf, o_hbm.at[pl.ds(off, _CHUNK)])
        return carry

    lax.fori_loop(0, _CHUNKS, chunk_body, 0)


def kernel(x):
    b, s, d = x.shape
    flat = x.reshape(_N)
    mesh = plsc.VectorSubcoreMesh(core_axis_name="c", subcore_axis_name="s")
    out = functools.partial(
        pl.kernel,
        mesh=mesh,
        out_type=jax.ShapeDtypeStruct((_N,), jnp.float32),
        scratch_types=[pltpu.VMEM((_CHUNK,), jnp.float32)],
    )(_sc_relu)(flat)
    return out.reshape(b, s, d)


# speedup vs baseline: 1.2282x; 1.2282x over previous
import functools

import jax
import jax.numpy as jnp
from jax import lax
from jax.experimental import pallas as pl
from jax.experimental.pallas import tpu as pltpu
from jax.experimental.pallas import tpu_sc as plsc

_info = plsc.get_sparse_core_info()
_NC, _NS, _L = _info.num_cores, _info.num_subcores, _info.num_lanes
_NW = _NC * _NS

_N = 4 * 4096 * 2048
_PER_W = _N // _NW
_CHUNK = 16384
_CHUNKS = _PER_W // _CHUNK


def _sc_relu(x_hbm, o_hbm, bin0, bin1, bout, si0, si1):
    wid = lax.axis_index("s") * _NC + lax.axis_index("c")
    base = wid * _PER_W
    bins = (bin0, bin1)
    sis = (si0, si1)

    pltpu.async_copy(x_hbm.at[pl.ds(base, _CHUNK)], bin0, si0)
    pltpu.async_copy(x_hbm.at[pl.ds(base + _CHUNK, _CHUNK)], bin1, si1)

    def gbody(g, carry):
        for b in range(2):
            ci = g * 2 + b
            off = base + ci * _CHUNK
            pltpu.make_async_copy(x_hbm.at[pl.ds(off, _CHUNK)], bins[b], sis[b]).wait()

            def vec(vi, c, b=b):
                s = pl.ds(vi * _L, _L)
                bout[s] = jnp.maximum(bins[b][s], 0.0)
                return c

            lax.fori_loop(0, _CHUNK // _L, vec, 0)

            pltpu.sync_copy(bout, o_hbm.at[pl.ds(off, _CHUNK)])

            @pl.when(ci + 2 < _CHUNKS)
            def _next_in(off=off, b=b):
                pltpu.async_copy(
                    x_hbm.at[pl.ds(off + 2 * _CHUNK, _CHUNK)], bins[b], sis[b]
                )
        return carry

    lax.fori_loop(0, _CHUNKS // 2, gbody, 0)


def kernel(x):
    b, s, d = x.shape
    flat = x.reshape(_N)
    mesh = plsc.VectorSubcoreMesh(core_axis_name="c", subcore_axis_name="s")
    out = functools.partial(
        pl.kernel,
        mesh=mesh,
        out_type=jax.ShapeDtypeStruct((_N,), jnp.float32),
        scratch_types=[
            pltpu.VMEM((_CHUNK,), jnp.float32),
            pltpu.VMEM((_CHUNK,), jnp.float32),
            pltpu.VMEM((_CHUNK,), jnp.float32),
            pltpu.SemaphoreType.DMA,
            pltpu.SemaphoreType.DMA,
        ],
    )(_sc_relu)(flat)
    return out.reshape(b, s, d)


# speedup vs baseline: 1.3275x; 1.0808x over previous
import functools

import jax
import jax.numpy as jnp
from jax import lax
from jax.experimental import pallas as pl
from jax.experimental.pallas import tpu as pltpu
from jax.experimental.pallas import tpu_sc as plsc

_info = plsc.get_sparse_core_info()
_NC, _NS, _L = _info.num_cores, _info.num_subcores, _info.num_lanes
_NW = _NC * _NS

_N = 4 * 4096 * 2048
_PER_W = _N // _NW
_CHUNK = 16384
_CHUNKS = _PER_W // _CHUNK


def _sc_relu(x_hbm, o_hbm, bin0, bin1, bout0, bout1, si0, si1, so0, so1):
    wid = lax.axis_index("s") * _NC + lax.axis_index("c")
    base = wid * _PER_W
    bins = (bin0, bin1)
    bouts = (bout0, bout1)
    sis = (si0, si1)
    sos = (so0, so1)

    pltpu.async_copy(x_hbm.at[pl.ds(base, _CHUNK)], bin0, si0)
    pltpu.async_copy(x_hbm.at[pl.ds(base + _CHUNK, _CHUNK)], bin1, si1)

    def gbody(g, carry):
        for b in range(2):
            ci = g * 2 + b
            off = base + ci * _CHUNK
            pltpu.make_async_copy(x_hbm.at[pl.ds(off, _CHUNK)], bins[b], sis[b]).wait()

            @pl.when(g > 0)
            def _drain_prev(off=off, b=b):
                prev = off - 2 * _CHUNK
                pltpu.make_async_copy(
                    bouts[b], o_hbm.at[pl.ds(prev, _CHUNK)], sos[b]
                ).wait()

            def vec(vi, c, b=b):
                s = pl.ds(vi * _L, _L)
                bouts[b][s] = jnp.maximum(bins[b][s], 0.0)
                return c

            lax.fori_loop(0, _CHUNK // _L, vec, 0)

            pltpu.async_copy(bouts[b], o_hbm.at[pl.ds(off, _CHUNK)], sos[b])

            @pl.when(ci + 2 < _CHUNKS)
            def _next_in(off=off, b=b):
                pltpu.async_copy(
                    x_hbm.at[pl.ds(off + 2 * _CHUNK, _CHUNK)], bins[b], sis[b]
                )
        return carry

    lax.fori_loop(0, _CHUNKS // 2, gbody, 0)

    for b in range(2):
        last = base + (_CHUNKS - 2 + b) * _CHUNK
        pltpu.make_async_copy(bouts[b], o_hbm.at[pl.ds(last, _CHUNK)], sos[b]).wait()


def kernel(x):
    b, s, d = x.shape
    flat = x.reshape(_N)
    mesh = plsc.VectorSubcoreMesh(core_axis_name="c", subcore_axis_name="s")
    out = functools.partial(
        pl.kernel,
        mesh=mesh,
        out_type=jax.ShapeDtypeStruct((_N,), jnp.float32),
        scratch_types=[
            pltpu.VMEM((_CHUNK,), jnp.float32),
            pltpu.VMEM((_CHUNK,), jnp.float32),
            pltpu.VMEM((_CHUNK,), jnp.float32),
            pltpu.VMEM((_CHUNK,), jnp.float32),
            pltpu.SemaphoreType.DMA,
            pltpu.SemaphoreType.DMA,
            pltpu.SemaphoreType.DMA,
            pltpu.SemaphoreType.DMA,
        ],
    )(_sc_relu)(flat)
    return out.reshape(b, s, d)


# speedup vs baseline: 1.9864x; 1.4964x over previous
import functools

import jax
import jax.numpy as jnp
from jax import lax
from jax.experimental import pallas as pl
from jax.experimental.pallas import tpu as pltpu
from jax.experimental.pallas import tpu_sc as plsc

_info = plsc.get_sparse_core_info()
_NC, _NS, _L = _info.num_cores, _info.num_subcores, _info.num_lanes
_NW = _NC * _NS

_N = 4 * 4096 * 2048
_PER_W = _N // _NW
_CHUNK = 16384
_CHUNKS = _PER_W // _CHUNK


def _sc_relu(x_hbm, o_hbm, bin0, bin1, bout0, bout1, si0, si1, so0, so1):
    wid = lax.axis_index("s") * _NC + lax.axis_index("c")
    base = wid * _PER_W
    bins = (bin0, bin1)
    bouts = (bout0, bout1)
    sis = (si0, si1)
    sos = (so0, so1)

    pltpu.async_copy(x_hbm.at[pl.ds(base, _CHUNK)], bin0, si0)
    pltpu.async_copy(x_hbm.at[pl.ds(base + _CHUNK, _CHUNK)], bin1, si1)

    def gbody(g, carry):
        for b in range(2):
            ci = g * 2 + b
            off = base + ci * _CHUNK
            pltpu.make_async_copy(x_hbm.at[pl.ds(off, _CHUNK)], bins[b], sis[b]).wait()

            @pl.when(g > 0)
            def _drain_prev(off=off, b=b):
                prev = off - 2 * _CHUNK
                pltpu.make_async_copy(
                    bouts[b], o_hbm.at[pl.ds(prev, _CHUNK)], sos[b]
                ).wait()

            def vec(vi, c, b=b):
                for k in range(8):
                    s = pl.ds(vi * (8 * _L) + k * _L, _L)
                    bouts[b][s] = jnp.maximum(bins[b][s], 0.0)
                return c

            lax.fori_loop(0, _CHUNK // (8 * _L), vec, 0)

            pltpu.async_copy(bouts[b], o_hbm.at[pl.ds(off, _CHUNK)], sos[b])

            @pl.when(ci + 2 < _CHUNKS)
            def _next_in(off=off, b=b):
                pltpu.async_copy(
                    x_hbm.at[pl.ds(off + 2 * _CHUNK, _CHUNK)], bins[b], sis[b]
                )
        return carry

    lax.fori_loop(0, _CHUNKS // 2, gbody, 0)

    for b in range(2):
        last = base + (_CHUNKS - 2 + b) * _CHUNK
        pltpu.make_async_copy(bouts[b], o_hbm.at[pl.ds(last, _CHUNK)], sos[b]).wait()


def kernel(x):
    b, s, d = x.shape
    flat = x.reshape(_N)
    mesh = plsc.VectorSubcoreMesh(core_axis_name="c", subcore_axis_name="s")
    out = functools.partial(
        pl.kernel,
        mesh=mesh,
        out_type=jax.ShapeDtypeStruct((_N,), jnp.float32),
        scratch_types=[
            pltpu.VMEM((_CHUNK,), jnp.float32),
            pltpu.VMEM((_CHUNK,), jnp.float32),
            pltpu.VMEM((_CHUNK,), jnp.float32),
            pltpu.VMEM((_CHUNK,), jnp.float32),
            pltpu.SemaphoreType.DMA,
            pltpu.SemaphoreType.DMA,
            pltpu.SemaphoreType.DMA,
            pltpu.SemaphoreType.DMA,
        ],
    )(_sc_relu)(flat)
    return out.reshape(b, s, d)


# speedup vs baseline: 2.0112x; 1.0125x over previous
import functools

import jax
import jax.numpy as jnp
from jax import lax
from jax.experimental import pallas as pl
from jax.experimental.pallas import tpu as pltpu
from jax.experimental.pallas import tpu_sc as plsc

_info = plsc.get_sparse_core_info()
_NC, _NS, _L = _info.num_cores, _info.num_subcores, _info.num_lanes
_NW = _NC * _NS

_N = 4 * 4096 * 2048
_PER_W = _N // _NW
_CHUNK = 24576
_CHUNKS = _PER_W // _CHUNK


def _sc_relu(x_hbm, o_hbm, bin0, bin1, bout0, bout1, si0, si1, so0, so1):
    wid = lax.axis_index("s") * _NC + lax.axis_index("c")
    base = wid * _PER_W
    bins = (bin0, bin1)
    bouts = (bout0, bout1)
    sis = (si0, si1)
    sos = (so0, so1)

    pltpu.async_copy(x_hbm.at[pl.ds(base, _CHUNK)], bin0, si0)
    pltpu.async_copy(x_hbm.at[pl.ds(base + _CHUNK, _CHUNK)], bin1, si1)

    def gbody(g, carry):
        for b in range(2):
            ci = g * 2 + b
            off = base + ci * _CHUNK
            pltpu.make_async_copy(x_hbm.at[pl.ds(off, _CHUNK)], bins[b], sis[b]).wait()

            @pl.when(g > 0)
            def _drain_prev(off=off, b=b):
                prev = off - 2 * _CHUNK
                pltpu.make_async_copy(
                    bouts[b], o_hbm.at[pl.ds(prev, _CHUNK)], sos[b]
                ).wait()

            def vec(vi, c, b=b):
                for k in range(8):
                    s = pl.ds(vi * (8 * _L) + k * _L, _L)
                    bouts[b][s] = jnp.maximum(bins[b][s], 0.0)
                return c

            lax.fori_loop(0, _CHUNK // (8 * _L), vec, 0)

            pltpu.async_copy(bouts[b], o_hbm.at[pl.ds(off, _CHUNK)], sos[b])

            @pl.when(ci + 2 < _CHUNKS)
            def _next_in(off=off, b=b):
                pltpu.async_copy(
                    x_hbm.at[pl.ds(off + 2 * _CHUNK, _CHUNK)], bins[b], sis[b]
                )
        return carry

    lax.fori_loop(0, _CHUNKS // 2, gbody, 0)

    for b in range(2):
        last = base + (_CHUNKS - 2 + b) * _CHUNK
        pltpu.make_async_copy(bouts[b], o_hbm.at[pl.ds(last, _CHUNK)], sos[b]).wait()


def kernel(x):
    b, s, d = x.shape
    flat = x.reshape(_N)
    mesh = plsc.VectorSubcoreMesh(core_axis_name="c", subcore_axis_name="s")
    out = functools.partial(
        pl.kernel,
        mesh=mesh,
        out_type=jax.ShapeDtypeStruct((_N,), jnp.float32),
        scratch_types=[
            pltpu.VMEM((_CHUNK,), jnp.float32),
            pltpu.VMEM((_CHUNK,), jnp.float32),
            pltpu.VMEM((_CHUNK,), jnp.float32),
            pltpu.VMEM((_CHUNK,), jnp.float32),
            pltpu.SemaphoreType.DMA,
            pltpu.SemaphoreType.DMA,
            pltpu.SemaphoreType.DMA,
            pltpu.SemaphoreType.DMA,
        ],
    )(_sc_relu)(flat)
    return out.reshape(b, s, d)


# speedup vs baseline: 8.2237x; 4.0890x over previous
import jax
import jax.numpy as jnp
from jax.experimental import pallas as pl
from jax.experimental.pallas import tpu as pltpu


def _relu_body(x_ref, o_ref):
    o_ref[...] = jnp.maximum(x_ref[...], 0.0)


def kernel(x):
    b, s, d = x.shape
    rows = b * s
    x2 = x.reshape(rows, d)
    block_rows = 1024
    grid = rows // block_rows
    out = pl.pallas_call(
        _relu_body,
        grid=(grid,),
        in_specs=[pl.BlockSpec((block_rows, d), lambda i: (i, 0))],
        out_specs=pl.BlockSpec((block_rows, d), lambda i: (i, 0)),
        out_shape=jax.ShapeDtypeStruct((rows, d), x.dtype),
        compiler_params=pltpu.CompilerParams(
            dimension_semantics=("arbitrary",),
        ),
    )(x2)
    return out.reshape(b, s, d)
